# baseline (speedup 1.0000x reference)
import functools

import jax
import jax.numpy as jnp
from jax import lax
from jax.experimental import pallas as pl
from jax.experimental.pallas import tpu as pltpu
from jax.experimental.pallas import tpu_sc as plsc

B = 1024
L = 50
VOCAB = 100000
HIDDEN = 64

NC = 2
NS = 16
NW = NC * NS
H_PER_W = HIDDEN // NW
NPIECE = 2
PIECE = VOCAB // NPIECE
PIECE_PAD = PIECE + 16
CHUNK = 128
NCHUNK = B // CHUNK
BG = CHUNK // 16
NTASK = H_PER_W * NPIECE


def _sc_body(words_hbm, w_hbm, b_hbm, out_hbm,
             words_sh, wbuf0, wbuf1, wc_v, outrow_v, bvec_v,
             semw0, semw1):
    cid = lax.axis_index("c")
    sid = lax.axis_index("s")
    wid = sid * NC + cid
    wbufs = [wbuf0, wbuf1]
    semws = [semw0, semw1]

    @pl.when(sid == 0)
    def _():
        pltpu.sync_copy(words_hbm, words_sh)

    pltpu.sync_copy(b_hbm, bvec_v.at[pl.ds(0, HIDDEN)])
    plsc.subcore_barrier()

    lanes = lax.iota(jnp.int32, 16)
    zeros16 = jnp.zeros((16,), jnp.float32)

    def issue_w(t):
        h = wid * H_PER_W + t // NPIECE
        vp = t % NPIECE
        return pltpu.async_copy(
            w_hbm.at[pl.ds(h * VOCAB + vp * PIECE, PIECE)],
            wbufs[t % 2].at[pl.ds(0, PIECE)],
            semws[t % 2],
        )

    wh = [issue_w(0), None]

    for t in range(NTASK):
        vp = t % NPIECE
        h = wid * H_PER_W + t // NPIECE
        if t + 1 < NTASK:
            wh[(t + 1) % 2] = issue_w(t + 1)
        wh[t % 2].wait()
        wbuf = wbufs[t % 2]
        wbuf[pl.ds(PIECE, 16)] = zeros16
        bh = plsc.load_gather(bvec_v, [jnp.full((16,), 0, jnp.int32) + h])

        def chunk_body(c, _, wbuf=wbuf, bh=bh, vp=vp):
            pltpu.sync_copy(
                words_sh.at[pl.ds(c * (CHUNK * L), CHUNK * L)], wc_v
            )

            def bg_body(g, _, wbuf=wbuf, bh=bh, vp=vp, c=c):
                base = (g * 16 + lanes) * L
                if vp == 0:
                    acc0 = bh
                else:
                    acc0 = outrow_v[pl.ds(c * CHUNK + g * 16, 16)]
                acc1 = zeros16
                for j in range(0, L, 2):
                    w0 = plsc.load_gather(wc_v, [base + j])
                    w1 = plsc.load_gather(wc_v, [base + (j + 1)])
                    i0 = w0 - (vp * PIECE)
                    i1 = w1 - (vp * PIECE)
                    i0 = plsc.bitcast(
                        jnp.minimum(plsc.bitcast(i0, jnp.uint32),
                                    jnp.uint32(PIECE)), jnp.int32)
                    i1 = plsc.bitcast(
                        jnp.minimum(plsc.bitcast(i1, jnp.uint32),
                                    jnp.uint32(PIECE)), jnp.int32)
                    acc0 = acc0 + plsc.load_gather(wbuf, [i0])
                    acc1 = acc1 + plsc.load_gather(wbuf, [i1])
                outrow_v[pl.ds(c * CHUNK + g * 16, 16)] = acc0 + acc1
                return 0

            lax.fori_loop(0, BG, bg_body, 0)
            return 0

        lax.fori_loop(0, NCHUNK, chunk_body, 0)

        if vp == NPIECE - 1:
            pltpu.sync_copy(outrow_v, out_hbm.at[h])


@functools.partial(jax.jit, donate_argnums=())
def _launch(words_flat, W, b):
    mesh = plsc.VectorSubcoreMesh(core_axis_name="c", subcore_axis_name="s")
    f = pl.kernel(
        _sc_body,
        out_type=jax.ShapeDtypeStruct((HIDDEN, B), jnp.float32),
        mesh=mesh,
        scratch_types=[
            pltpu.VMEM_SHARED((B * L,), jnp.int32),
            pltpu.VMEM((PIECE_PAD,), jnp.float32),
            pltpu.VMEM((PIECE_PAD,), jnp.float32),
            pltpu.VMEM((CHUNK * L,), jnp.int32),
            pltpu.VMEM((B,), jnp.float32),
            pltpu.VMEM((128,), jnp.float32),
            pltpu.SemaphoreType.DMA,
            pltpu.SemaphoreType.DMA,
        ],
        compiler_params=pltpu.CompilerParams(needs_layout_passes=False),
    )
    return f(words_flat, W, b)


def kernel(words, W, b):
    words_flat = words.reshape(-1).astype(jnp.int32)
    out_t = _launch(words_flat, W.reshape(-1), b)
    return out_t.T


# speedup vs baseline: 1.2540x; 1.2540x over previous
import functools

import jax
import jax.numpy as jnp
from jax import lax
from jax.experimental import pallas as pl
from jax.experimental.pallas import tpu as pltpu
from jax.experimental.pallas import tpu_sc as plsc

B = 1024
L = 50
VOCAB = 100000
HIDDEN = 64

NC = 2
NS = 16
NW = NC * NS
H_PER_W = HIDDEN // NW
CHUNK = 128
NCHUNK = B // CHUNK
BG = CHUNK // 16


def _sc_body(words_hbm, w_hbm, b_hbm, out_hbm,
             wrow_v, wc0, wc1, outrow_v, bvec_v, semc0, semc1, semw):
    cid = lax.axis_index("c")
    sid = lax.axis_index("s")
    wid = sid * NC + cid
    wcs = [wc0, wc1]
    semcs = [semc0, semc1]

    pltpu.sync_copy(b_hbm, bvec_v.at[pl.ds(0, HIDDEN)])
    lanes = lax.iota(jnp.int32, 16)
    zeros16 = jnp.zeros((16,), jnp.float32)

    def words_copy(c, buf):
        return pltpu.make_async_copy(
            words_hbm.at[pl.ds(c * (CHUNK * L), CHUNK * L)],
            wcs[buf],
            semcs[buf],
        )

    def w_copy(h):
        return pltpu.make_async_copy(
            w_hbm.at[pl.ds(h * VOCAB, VOCAB)],
            wrow_v,
            semw,
        )

    w_copy(wid * H_PER_W).start()

    for hi in range(H_PER_W):
        h = wid * H_PER_W + hi
        w_copy(h).wait()
        bh = plsc.load_gather(bvec_v, [jnp.full((16,), 0, jnp.int32) + h])
        words_copy(jnp.int32(0), 0).start()
        words_copy(jnp.int32(1), 1).start()

        def pair_body(cc, _, bh=bh):
            c0 = cc * 2
            for par in range(2):
                c = c0 + par
                words_copy(c, par).wait()
                wordsc = wcs[par]

                def bg_body(g, _, wordsc=wordsc, bh=bh, c=c):
                    base = (g * 16 + lanes) * L
                    acc0 = bh
                    acc1 = zeros16
                    for j in range(0, L, 2):
                        w0 = plsc.load_gather(wordsc, [base + j])
                        w1 = plsc.load_gather(wordsc, [base + (j + 1)])
                        acc0 = acc0 + plsc.load_gather(wrow_v, [w0])
                        acc1 = acc1 + plsc.load_gather(wrow_v, [w1])
                    outrow_v[pl.ds(c * CHUNK + g * 16, 16)] = acc0 + acc1
                    return 0

                lax.fori_loop(0, BG, bg_body, 0)

                @pl.when(c + 2 < NCHUNK)
                def _(c=c, par=par):
                    words_copy(c + 2, par).start()

            return 0

        lax.fori_loop(0, NCHUNK // 2, pair_body, 0)

        if hi + 1 < H_PER_W:
            w_copy(h + 1).start()
        pltpu.sync_copy(outrow_v, out_hbm.at[h])


@functools.partial(jax.jit, donate_argnums=())
def _launch(words_flat, w_flat, b):
    mesh = plsc.VectorSubcoreMesh(core_axis_name="c", subcore_axis_name="s")
    f = pl.kernel(
        _sc_body,
        out_type=jax.ShapeDtypeStruct((HIDDEN, B), jnp.float32),
        mesh=mesh,
        scratch_types=[
            pltpu.VMEM((VOCAB,), jnp.float32),
            pltpu.VMEM((CHUNK * L,), jnp.int32),
            pltpu.VMEM((CHUNK * L,), jnp.int32),
            pltpu.VMEM((B,), jnp.float32),
            pltpu.VMEM((128,), jnp.float32),
            pltpu.SemaphoreType.DMA,
            pltpu.SemaphoreType.DMA,
            pltpu.SemaphoreType.DMA,
        ],
        compiler_params=pltpu.CompilerParams(needs_layout_passes=False),
    )
    return f(words_flat, w_flat, b)


def kernel(words, W, b):
    words_flat = words.reshape(-1).astype(jnp.int32)
    out_t = _launch(words_flat, W.reshape(-1), b)
    return out_t.T
